# baseline (speedup 1.0000x reference)
import functools

import jax
import jax.numpy as jnp
from jax import lax
from jax.experimental import pallas as pl
from jax.experimental.pallas import tpu as pltpu
from jax.experimental.pallas import tpu_sc as plsc

_DIM = 32
_LANES = 16


def _make_kernel(batch):
    info = plsc.get_sparse_core_info()
    nc, ns = info.num_cores, info.num_subcores
    nw = nc * ns
    b_per_w = batch // nw
    n_chunks = b_per_w // 128
    n_groups = b_per_w // _LANES
    groups_per_chunk = 128 // _LANES

    mesh = plsc.VectorSubcoreMesh(core_axis_name="c", subcore_axis_name="s")

    @functools.partial(
        pl.kernel,
        mesh=mesh,
        out_type=jax.ShapeDtypeStruct((nw, _LANES), jnp.float32),
        compiler_params=pltpu.CompilerParams(
            needs_layout_passes=False, use_tc_tiling_on_sc=False),
        scratch_types=[
            pltpu.VMEM((n_chunks, 128), jnp.int32),
            pltpu.VMEM((n_chunks, 128), jnp.int32),
            pltpu.VMEM((b_per_w,), jnp.float32),
            pltpu.VMEM((b_per_w, _DIM), jnp.float32),
            pltpu.VMEM((b_per_w, _DIM), jnp.float32),
            pltpu.VMEM((_LANES,), jnp.float32),
            pltpu.SemaphoreType.DMA,
        ],
    )
    def sc_kernel(left_hbm, right_hbm, cov_hbm, table_hbm, out_hbm,
                  lidx_v, ridx_v, cov_v, lrows_v, rrows_v, loss_v, sem):
        wid = lax.axis_index("s") * nc + lax.axis_index("c")

        pltpu.sync_copy(left_hbm.at[wid], lidx_v)
        pltpu.sync_copy(right_hbm.at[wid], ridx_v)
        pltpu.sync_copy(cov_hbm.at[wid], cov_v)

        copies = []
        for j in range(n_chunks):
            copies.append(pltpu.async_copy(
                table_hbm.at[lidx_v.at[j]],
                lrows_v.at[pl.ds(j * 128, 128)], sem))
            copies.append(pltpu.async_copy(
                table_hbm.at[ridx_v.at[j]],
                rrows_v.at[pl.ds(j * 128, 128)], sem))
        for c in copies:
            c.wait()

        lane = lax.iota(jnp.int32, _LANES)

        def group_body(g, loss):
            row = g * _LANES + lane
            acc = jnp.zeros((_LANES,), jnp.float32)
            for c in range(_DIM):
                col = jnp.full((_LANES,), c, jnp.int32)
                lv = plsc.load_gather(lrows_v, [row, col])
                rv = plsc.load_gather(rrows_v, [row, col])
                acc = acc + lv * rv
            d = acc - cov_v[pl.ds(g * _LANES, _LANES)]
            return loss + d * d

        loss = lax.fori_loop(0, n_groups, group_body,
                             jnp.zeros((_LANES,), jnp.float32))
        loss_v[...] = loss
        pltpu.sync_copy(loss_v, out_hbm.at[wid])

    return nw, n_chunks, sc_kernel


def kernel(left, right, covariances, table):
    batch = left.shape[0]
    nw, n_chunks, sc_kernel = _make_kernel(batch)
    left3 = left.astype(jnp.int32).reshape(nw, n_chunks, 128)
    right3 = right.astype(jnp.int32).reshape(nw, n_chunks, 128)
    cov2 = covariances.reshape(nw, batch // nw)
    partials = sc_kernel(left3, right3, cov2, table)
    return jnp.sum(partials) / batch
